# baseline (speedup 1.0000x reference)
import functools

import jax
import jax.numpy as jnp
from jax import lax
from jax.experimental import pallas as pl
from jax.experimental.pallas import tpu as pltpu
from jax.experimental.pallas import tpu_sc as plsc

N = 10000
E = 320000
D = 128
NP = 10240
NC = 2
NS = 16
NW = NC * NS
CH = 128
NCHUNK = 80
EPT = NCHUNK * CH
EP = NW * EPT
ROWS = NP // NS
BLK = 512
NBLK = NP // BLK

_mesh = plsc.VectorSubcoreMesh(core_axis_name="c", subcore_axis_name="s")


@functools.partial(
    pl.kernel,
    out_type=(
        jax.ShapeDtypeStruct((NW, NP), jnp.float32),
        jax.ShapeDtypeStruct((NW, NP), jnp.float32),
    ),
    mesh=_mesh,
    scratch_types=[
        pltpu.VMEM((NCHUNK, CH), jnp.int32),
        pltpu.VMEM((NP,), jnp.float32),
        pltpu.VMEM((NP,), jnp.float32),
    ],
    compiler_params=pltpu.CompilerParams(needs_layout_passes=False),
)
def _hist(src3, dst3, zN, degs_out, degd_out, idx_v, cs, cd):
    c = lax.axis_index("c")
    s = lax.axis_index("s")
    g = s * NC + c
    pltpu.sync_copy(zN, cs)
    pltpu.sync_copy(zN, cd)
    ones = jnp.ones((16,), jnp.float32)

    def count_into(cnt):
        def body(j, carry):
            def inner(k, carry2):
                iv = idx_v[j, pl.ds(k * 16, 16)]
                plsc.addupdate_scatter(cnt, [iv], ones)
                return carry2
            return lax.fori_loop(0, CH // 16, inner, carry)
        lax.fori_loop(0, NCHUNK, body, 0)

    pltpu.sync_copy(src3.at[g], idx_v)
    count_into(cs)
    pltpu.sync_copy(dst3.at[g], idx_v)
    count_into(cd)

    pltpu.sync_copy(cs, degs_out.at[g])
    pltpu.sync_copy(cd, degd_out.at[g])


def _mm_body(feat_ref, w_ref, x_ref):
    x_ref[...] = jnp.dot(feat_ref[...], w_ref[...],
                         preferred_element_type=jnp.float32)


def _mm(featp, W):
    return pl.pallas_call(
        _mm_body,
        grid=(NBLK,),
        in_specs=[
            pl.BlockSpec((BLK, D), lambda i: (i, 0)),
            pl.BlockSpec((D, D), lambda i: (0, 0)),
        ],
        out_specs=pl.BlockSpec((BLK, D), lambda i: (i, 0)),
        out_shape=jax.ShapeDtypeStruct((NP, D), jnp.float32),
    )(featp, W)


def _scale_body(xr_ref, degs_ref, x_ref):
    d = jnp.sum(degs_ref[...], axis=0)[:, None]
    ns = jnp.where(d > 0, lax.rsqrt(d), 0.0)
    x_ref[...] = xr_ref[...] * ns


def _scale(xr, degs):
    return pl.pallas_call(
        _scale_body,
        grid=(NBLK,),
        in_specs=[
            pl.BlockSpec((BLK, D), lambda i: (i, 0)),
            pl.BlockSpec((NW, BLK), lambda i: (0, i)),
        ],
        out_specs=pl.BlockSpec((BLK, D), lambda i: (i, 0)),
        out_shape=jax.ShapeDtypeStruct((NP, D), jnp.float32),
    )(xr, degs)


ST = 40
NCH0 = 80
NCH1 = 80


@functools.partial(
    pl.kernel,
    out_type=jax.ShapeDtypeStruct((NC, NP, D), jnp.float32),
    mesh=_mesh,
    scratch_types=[
        pltpu.VMEM((ST, CH), jnp.int32),
        pltpu.VMEM((ST, CH), jnp.int32),
        pltpu.VMEM((CH, D), jnp.float32),
        pltpu.VMEM((CH, D), jnp.float32),
        pltpu.VMEM_SHARED((NP, D), jnp.float32),
        pltpu.SemaphoreType.DMA,
        pltpu.SemaphoreType.DMA,
    ],
)
def _agg(x, src3a, dst3a, src3b, dst3b, out,
         sidx, didx, buf0, buf1, agg_sh, sem0, sem1):
    c = lax.axis_index("c")
    s = lax.axis_index("s")
    pltpu.sync_copy(x.at[pl.ds(N, CH)], buf0)
    for k in range(ROWS // CH):
        pltpu.sync_copy(buf0, agg_sh.at[pl.ds(s * ROWS + k * CH, CH)])
    plsc.subcore_barrier()

    def run_phases(srcR, dstR, nch):
        for p in range(nch // ST):
            pltpu.sync_copy(srcR.at[s, pl.ds(p * ST, ST)], sidx)
            pltpu.sync_copy(dstR.at[s, pl.ds(p * ST, ST)], didx)

            pltpu.async_copy(x.at[sidx.at[0]], buf0, sem0)

            def body(t, carry):
                j0 = 2 * t
                pltpu.async_copy(x.at[sidx.at[j0 + 1]], buf1, sem1)
                pltpu.make_async_copy(x.at[sidx.at[j0]], buf0, sem0).wait()
                pltpu.sync_copy(buf0, agg_sh.at[didx.at[j0]], add=True)
                pltpu.async_copy(x.at[sidx.at[j0 + 2]], buf0, sem0)
                pltpu.make_async_copy(x.at[sidx.at[j0 + 1]], buf1, sem1).wait()
                pltpu.sync_copy(buf1, agg_sh.at[didx.at[j0 + 1]], add=True)
                return carry

            lax.fori_loop(0, ST // 2 - 1, body, 0)

            jl = ST - 2
            pltpu.async_copy(x.at[sidx.at[jl + 1]], buf1, sem1)
            pltpu.make_async_copy(x.at[sidx.at[jl]], buf0, sem0).wait()
            pltpu.sync_copy(buf0, agg_sh.at[didx.at[jl]], add=True)
            pltpu.make_async_copy(x.at[sidx.at[jl + 1]], buf1, sem1).wait()
            pltpu.sync_copy(buf1, agg_sh.at[didx.at[jl + 1]], add=True)

    if NCH0 > 0:
        @pl.when(c == 0)
        def _c0():
            run_phases(src3a, dst3a, NCH0)

    @pl.when(c == 1)
    def _c1():
        run_phases(src3b, dst3b, NCH1)

    plsc.subcore_barrier()
    for k in range(ROWS // CH):
        so = s * ROWS + k * CH
        pltpu.sync_copy(agg_sh.at[pl.ds(so, CH)], out.at[c, pl.ds(so, CH)])


def _final_body(aggp_ref, degd_ref, b_ref, a_ref, h_ref, hg_ref):
    i = pl.program_id(0)
    agg = aggp_ref[0] + aggp_ref[1]
    d = jnp.sum(degd_ref[...], axis=0)[:, None]
    nd = jnp.where(d > 0, lax.rsqrt(d), 0.0)
    rst = agg * nd + b_ref[...]
    a = a_ref[0, 0]
    h = jnp.maximum(rst, 0.0) + a * jnp.minimum(rst, 0.0)
    row = i * BLK + lax.broadcasted_iota(jnp.int32, (BLK, 1), 0)
    h = jnp.where(row < N, h, 0.0)
    h_ref[...] = h

    @pl.when(i == 0)
    def _init():
        hg_ref[...] = jnp.zeros_like(hg_ref)

    hg_ref[...] += jnp.sum(h, axis=0, keepdims=True)


def _final(aggp, degd, b2, a2):
    return pl.pallas_call(
        _final_body,
        grid=(NBLK,),
        in_specs=[
            pl.BlockSpec((NC, BLK, D), lambda i: (0, i, 0)),
            pl.BlockSpec((NW, BLK), lambda i: (0, i)),
            pl.BlockSpec((1, D), lambda i: (0, 0)),
            pl.BlockSpec((1, 1), lambda i: (0, 0)),
        ],
        out_specs=[
            pl.BlockSpec((BLK, D), lambda i: (i, 0)),
            pl.BlockSpec((1, D), lambda i: (0, 0)),
        ],
        out_shape=[
            jax.ShapeDtypeStruct((NP, D), jnp.float32),
            jax.ShapeDtypeStruct((1, D), jnp.float32),
        ],
    )(aggp, degd, b2, a2)


def kernel(feat, edge_index, W, b, prelu_a):
    src = edge_index[0].astype(jnp.int32)
    dst = edge_index[1].astype(jnp.int32)
    fill = jnp.full((EP - E,), N, jnp.int32)
    src_pad = jnp.concatenate([src, fill])
    dst_pad = jnp.concatenate([dst, fill])
    src3 = src_pad.reshape(NW, NCHUNK, CH)
    dst3 = dst_pad.reshape(NW, NCHUNK, CH)
    ea = NS * NCH0 * CH
    if NCH0 > 0:
        src3a = src_pad[:ea].reshape(NS, NCH0, CH)
        dst3a = dst_pad[:ea].reshape(NS, NCH0, CH)
    else:
        src3a = jnp.zeros((NS, 1, CH), jnp.int32)
        dst3a = src3a
    src3b = src_pad[ea:].reshape(NS, NCH1, CH)
    dst3b = dst_pad[ea:].reshape(NS, NCH1, CH)
    featp = jnp.pad(feat, ((0, NP - N), (0, 0)))

    zN = jnp.zeros((NP,), jnp.float32)

    xr = _mm(featp, W)
    degs, degd = _hist(src3, dst3, zN)
    x = _scale(xr, degs)
    aggp = _agg(x, src3a, dst3a, src3b, dst3b)
    h_pad, hg = _final(aggp, degd, b.reshape(1, D),
                       jnp.reshape(prelu_a, (1, 1)))
    return h_pad[:N], hg


# speedup vs baseline: 1.0128x; 1.0128x over previous
import functools

import jax
import jax.numpy as jnp
from jax import lax
from jax.experimental import pallas as pl
from jax.experimental.pallas import tpu as pltpu
from jax.experimental.pallas import tpu_sc as plsc

N = 10000
E = 320000
D = 128
NP = 10240
NC = 2
NS = 16
NW = NC * NS
CH = 128
NCHUNK = 80
EPT = NCHUNK * CH
EP = NW * EPT
ROWS = NP // NS
BLK = 512
NBLK = NP // BLK

_mesh = plsc.VectorSubcoreMesh(core_axis_name="c", subcore_axis_name="s")


@functools.partial(
    pl.kernel,
    out_type=(
        jax.ShapeDtypeStruct((NW, NP), jnp.float32),
        jax.ShapeDtypeStruct((NW, NP), jnp.float32),
    ),
    mesh=_mesh,
    scratch_types=[
        pltpu.VMEM((NCHUNK, CH), jnp.int32),
        pltpu.VMEM((NP,), jnp.float32),
        pltpu.VMEM((NP,), jnp.float32),
    ],
    compiler_params=pltpu.CompilerParams(needs_layout_passes=False),
)
def _hist(src3, dst3, zN, degs_out, degd_out, idx_v, cs, cd):
    c = lax.axis_index("c")
    s = lax.axis_index("s")
    g = s * NC + c
    pltpu.sync_copy(zN, cs)
    pltpu.sync_copy(zN, cd)
    ones = jnp.ones((16,), jnp.float32)

    def count_into(cnt):
        def body(j, carry):
            def inner(k, carry2):
                iv = idx_v[j, pl.ds(k * 16, 16)]
                plsc.addupdate_scatter(cnt, [iv], ones)
                return carry2
            return lax.fori_loop(0, CH // 16, inner, carry)
        lax.fori_loop(0, NCHUNK, body, 0)

    pltpu.sync_copy(src3.at[g], idx_v)
    count_into(cs)
    pltpu.sync_copy(dst3.at[g], idx_v)
    count_into(cd)

    pltpu.sync_copy(cs, degs_out.at[g])
    pltpu.sync_copy(cd, degd_out.at[g])


def _xw_body(feat_ref, degs_ref, w_ref, x_ref):
    d = jnp.sum(degs_ref[...], axis=0)[:, None]
    ns = jnp.where(d > 0, lax.rsqrt(d), 0.0)
    x_ref[...] = jnp.dot(feat_ref[...] * ns, w_ref[...],
                         preferred_element_type=jnp.float32)


def _xw(featp, degs, W):
    return pl.pallas_call(
        _xw_body,
        grid=(NBLK,),
        in_specs=[
            pl.BlockSpec((BLK, D), lambda i: (i, 0)),
            pl.BlockSpec((NW, BLK), lambda i: (0, i)),
            pl.BlockSpec((D, D), lambda i: (0, 0)),
        ],
        out_specs=pl.BlockSpec((BLK, D), lambda i: (i, 0)),
        out_shape=jax.ShapeDtypeStruct((NP, D), jnp.float32),
    )(featp, degs, W)


ST = 40
NCH0 = 80
NCH1 = 80


@functools.partial(
    pl.kernel,
    out_type=jax.ShapeDtypeStruct((NC, NP, D), jnp.float32),
    mesh=_mesh,
    scratch_types=[
        pltpu.VMEM((ST, CH), jnp.int32),
        pltpu.VMEM((ST, CH), jnp.int32),
        pltpu.VMEM((CH, D), jnp.float32),
        pltpu.VMEM((CH, D), jnp.float32),
        pltpu.VMEM_SHARED((NP, D), jnp.float32),
        pltpu.SemaphoreType.DMA,
        pltpu.SemaphoreType.DMA,
    ],
)
def _agg(x, src3a, dst3a, src3b, dst3b, out,
         sidx, didx, buf0, buf1, agg_sh, sem0, sem1):
    c = lax.axis_index("c")
    s = lax.axis_index("s")
    pltpu.sync_copy(x.at[pl.ds(N, CH)], buf0)
    for k in range(ROWS // CH):
        pltpu.sync_copy(buf0, agg_sh.at[pl.ds(s * ROWS + k * CH, CH)])
    plsc.subcore_barrier()

    def run_phases(srcR, dstR, nch):
        for p in range(nch // ST):
            pltpu.sync_copy(srcR.at[s, pl.ds(p * ST, ST)], sidx)
            pltpu.sync_copy(dstR.at[s, pl.ds(p * ST, ST)], didx)

            pltpu.async_copy(x.at[sidx.at[0]], buf0, sem0)

            def body(t, carry):
                j0 = 2 * t
                pltpu.async_copy(x.at[sidx.at[j0 + 1]], buf1, sem1)
                pltpu.make_async_copy(x.at[sidx.at[j0]], buf0, sem0).wait()
                pltpu.sync_copy(buf0, agg_sh.at[didx.at[j0]], add=True)
                pltpu.async_copy(x.at[sidx.at[j0 + 2]], buf0, sem0)
                pltpu.make_async_copy(x.at[sidx.at[j0 + 1]], buf1, sem1).wait()
                pltpu.sync_copy(buf1, agg_sh.at[didx.at[j0 + 1]], add=True)
                return carry

            lax.fori_loop(0, ST // 2 - 1, body, 0)

            jl = ST - 2
            pltpu.async_copy(x.at[sidx.at[jl + 1]], buf1, sem1)
            pltpu.make_async_copy(x.at[sidx.at[jl]], buf0, sem0).wait()
            pltpu.sync_copy(buf0, agg_sh.at[didx.at[jl]], add=True)
            pltpu.make_async_copy(x.at[sidx.at[jl + 1]], buf1, sem1).wait()
            pltpu.sync_copy(buf1, agg_sh.at[didx.at[jl + 1]], add=True)

    if NCH0 > 0:
        @pl.when(c == 0)
        def _c0():
            run_phases(src3a, dst3a, NCH0)

    @pl.when(c == 1)
    def _c1():
        run_phases(src3b, dst3b, NCH1)

    plsc.subcore_barrier()
    for k in range(ROWS // CH):
        so = s * ROWS + k * CH
        pltpu.sync_copy(agg_sh.at[pl.ds(so, CH)], out.at[c, pl.ds(so, CH)])


def _final_body(aggp_ref, degd_ref, b_ref, a_ref, h_ref, hg_ref):
    i = pl.program_id(0)
    agg = aggp_ref[0] + aggp_ref[1]
    d = jnp.sum(degd_ref[...], axis=0)[:, None]
    nd = jnp.where(d > 0, lax.rsqrt(d), 0.0)
    rst = agg * nd + b_ref[...]
    a = a_ref[0, 0]
    h = jnp.maximum(rst, 0.0) + a * jnp.minimum(rst, 0.0)
    row = i * BLK + lax.broadcasted_iota(jnp.int32, (BLK, 1), 0)
    h = jnp.where(row < N, h, 0.0)
    h_ref[...] = h

    @pl.when(i == 0)
    def _init():
        hg_ref[...] = jnp.zeros_like(hg_ref)

    hg_ref[...] += jnp.sum(h, axis=0, keepdims=True)


def _final(aggp, degd, b2, a2):
    return pl.pallas_call(
        _final_body,
        grid=(NBLK,),
        in_specs=[
            pl.BlockSpec((NC, BLK, D), lambda i: (0, i, 0)),
            pl.BlockSpec((NW, BLK), lambda i: (0, i)),
            pl.BlockSpec((1, D), lambda i: (0, 0)),
            pl.BlockSpec((1, 1), lambda i: (0, 0)),
        ],
        out_specs=[
            pl.BlockSpec((BLK, D), lambda i: (i, 0)),
            pl.BlockSpec((1, D), lambda i: (0, 0)),
        ],
        out_shape=[
            jax.ShapeDtypeStruct((NP, D), jnp.float32),
            jax.ShapeDtypeStruct((1, D), jnp.float32),
        ],
    )(aggp, degd, b2, a2)


def kernel(feat, edge_index, W, b, prelu_a):
    src = edge_index[0].astype(jnp.int32)
    dst = edge_index[1].astype(jnp.int32)
    fill = jnp.full((EP - E,), N, jnp.int32)
    src_pad = jnp.concatenate([src, fill])
    dst_pad = jnp.concatenate([dst, fill])
    src3 = src_pad.reshape(NW, NCHUNK, CH)
    dst3 = dst_pad.reshape(NW, NCHUNK, CH)
    ea = NS * NCH0 * CH
    if NCH0 > 0:
        src3a = src_pad[:ea].reshape(NS, NCH0, CH)
        dst3a = dst_pad[:ea].reshape(NS, NCH0, CH)
    else:
        src3a = jnp.zeros((NS, 1, CH), jnp.int32)
        dst3a = src3a
    src3b = src_pad[ea:].reshape(NS, NCH1, CH)
    dst3b = dst_pad[ea:].reshape(NS, NCH1, CH)
    featp = jnp.pad(feat, ((0, NP - N), (0, 0)))

    zN = jnp.zeros((NP,), jnp.float32)

    degs, degd = _hist(src3, dst3, zN)
    x = _xw(featp, degs, W)
    aggp = _agg(x, src3a, dst3a, src3b, dst3b)
    h_pad, hg = _final(aggp, degd, b.reshape(1, D),
                       jnp.reshape(prelu_a, (1, 1)))
    return h_pad[:N], hg


# speedup vs baseline: 1.2314x; 1.2159x over previous
import functools

import jax
import jax.numpy as jnp
from jax import lax
from jax.experimental import pallas as pl
from jax.experimental.pallas import tpu as pltpu
from jax.experimental.pallas import tpu_sc as plsc

N = 10000
E = 320000
D = 128
NP = 10240
NC = 2
NS = 16
NW = NC * NS
CH = 128
NCHUNK = 80
EPT = NCHUNK * CH
EP = NW * EPT
ROWS = NP // NS
BLK = 512
NBLK = NP // BLK

_mesh = plsc.VectorSubcoreMesh(core_axis_name="c", subcore_axis_name="s")


@functools.partial(
    pl.kernel,
    out_type=(
        jax.ShapeDtypeStruct((NW, NP), jnp.float32),
        jax.ShapeDtypeStruct((NW, NP), jnp.float32),
    ),
    mesh=_mesh,
    scratch_types=[
        pltpu.VMEM((NCHUNK, CH), jnp.int32),
        pltpu.VMEM((NP,), jnp.float32),
        pltpu.VMEM((NP,), jnp.float32),
    ],
    compiler_params=pltpu.CompilerParams(needs_layout_passes=False),
)
def _hist(src3, dst3, zN, degs_out, degd_out, idx_v, cs, cd):
    c = lax.axis_index("c")
    s = lax.axis_index("s")
    g = s * NC + c
    pltpu.sync_copy(zN, cs)
    pltpu.sync_copy(zN, cd)
    ones = jnp.ones((16,), jnp.float32)

    def count_into(cnt):
        def body(j, carry):
            def inner(k, carry2):
                iv = idx_v[j, pl.ds(k * 16, 16)]
                plsc.addupdate_scatter(cnt, [iv], ones)
                return carry2
            return lax.fori_loop(0, CH // 16, inner, carry)
        lax.fori_loop(0, NCHUNK, body, 0)

    pltpu.sync_copy(src3.at[g], idx_v)
    count_into(cs)
    pltpu.sync_copy(dst3.at[g], idx_v)
    count_into(cd)

    pltpu.sync_copy(cs, degs_out.at[g])
    pltpu.sync_copy(cd, degd_out.at[g])


def _xw_body(feat_ref, degs_ref, w_ref, x_ref):
    d = jnp.sum(degs_ref[...], axis=0)[:, None]
    ns = jnp.where(d > 0, lax.rsqrt(d), 0.0)
    x_ref[...] = jnp.dot(feat_ref[...] * ns, w_ref[...],
                         preferred_element_type=jnp.float32)


def _xw(featp, degs, W):
    return pl.pallas_call(
        _xw_body,
        grid=(NBLK,),
        in_specs=[
            pl.BlockSpec((BLK, D), lambda i: (i, 0)),
            pl.BlockSpec((NW, BLK), lambda i: (0, i)),
            pl.BlockSpec((D, D), lambda i: (0, 0)),
        ],
        out_specs=pl.BlockSpec((BLK, D), lambda i: (i, 0)),
        out_shape=jax.ShapeDtypeStruct((NP, D), jnp.float32),
    )(featp, degs, W)


ST = 40


@functools.partial(
    pl.kernel,
    out_type=jax.ShapeDtypeStruct((NC, NP, D), jnp.float32),
    mesh=_mesh,
    scratch_types=[
        pltpu.VMEM((ST, CH), jnp.int32),
        pltpu.VMEM((ST, CH), jnp.int32),
        pltpu.VMEM((CH, D), jnp.float32),
        pltpu.VMEM((CH, D), jnp.float32),
        pltpu.VMEM_SHARED((NP, D), jnp.float32),
        pltpu.SemaphoreType.DMA,
        pltpu.SemaphoreType.DMA,
    ],
)
def _agg(x, src3, dst3, out,
         sidx, didx, buf0, buf1, agg_sh, sem0, sem1):
    c = lax.axis_index("c")
    s = lax.axis_index("s")
    g = s * NC + c
    pltpu.sync_copy(x.at[pl.ds(N, CH)], buf0)
    for k in range(ROWS // CH):
        pltpu.sync_copy(buf0, agg_sh.at[pl.ds(s * ROWS + k * CH, CH)])
    plsc.subcore_barrier()

    for p in range(NCHUNK // ST):
        pltpu.sync_copy(src3.at[g, pl.ds(p * ST, ST)], sidx)
        pltpu.sync_copy(dst3.at[g, pl.ds(p * ST, ST)], didx)

        pltpu.async_copy(x.at[sidx.at[0]], buf0, sem0)

        def body(t, carry):
            j0 = 2 * t
            pltpu.async_copy(x.at[sidx.at[j0 + 1]], buf1, sem1)
            pltpu.make_async_copy(x.at[sidx.at[j0]], buf0, sem0).wait()
            pltpu.sync_copy(buf0, agg_sh.at[didx.at[j0]], add=True)
            pltpu.async_copy(x.at[sidx.at[j0 + 2]], buf0, sem0)
            pltpu.make_async_copy(x.at[sidx.at[j0 + 1]], buf1, sem1).wait()
            pltpu.sync_copy(buf1, agg_sh.at[didx.at[j0 + 1]], add=True)
            return carry

        lax.fori_loop(0, ST // 2 - 1, body, 0)

        jl = ST - 2
        pltpu.async_copy(x.at[sidx.at[jl + 1]], buf1, sem1)
        pltpu.make_async_copy(x.at[sidx.at[jl]], buf0, sem0).wait()
        pltpu.sync_copy(buf0, agg_sh.at[didx.at[jl]], add=True)
        pltpu.make_async_copy(x.at[sidx.at[jl + 1]], buf1, sem1).wait()
        pltpu.sync_copy(buf1, agg_sh.at[didx.at[jl + 1]], add=True)

    plsc.subcore_barrier()
    for k in range(ROWS // CH):
        so = s * ROWS + k * CH
        pltpu.sync_copy(agg_sh.at[pl.ds(so, CH)], out.at[c, pl.ds(so, CH)])


def _final_body(aggp_ref, degd_ref, b_ref, a_ref, h_ref, hg_ref):
    i = pl.program_id(0)
    agg = aggp_ref[0] + aggp_ref[1]
    d = jnp.sum(degd_ref[...], axis=0)[:, None]
    nd = jnp.where(d > 0, lax.rsqrt(d), 0.0)
    rst = agg * nd + b_ref[...]
    a = a_ref[0, 0]
    h = jnp.maximum(rst, 0.0) + a * jnp.minimum(rst, 0.0)
    row = i * BLK + lax.broadcasted_iota(jnp.int32, (BLK, 1), 0)
    h = jnp.where(row < N, h, 0.0)
    h_ref[...] = h

    @pl.when(i == 0)
    def _init():
        hg_ref[...] = jnp.zeros_like(hg_ref)

    hg_ref[...] += jnp.sum(h, axis=0, keepdims=True)


def _final(aggp, degd, b2, a2):
    return pl.pallas_call(
        _final_body,
        grid=(NBLK,),
        in_specs=[
            pl.BlockSpec((NC, BLK, D), lambda i: (0, i, 0)),
            pl.BlockSpec((NW, BLK), lambda i: (0, i)),
            pl.BlockSpec((1, D), lambda i: (0, 0)),
            pl.BlockSpec((1, 1), lambda i: (0, 0)),
        ],
        out_specs=[
            pl.BlockSpec((BLK, D), lambda i: (i, 0)),
            pl.BlockSpec((1, D), lambda i: (0, 0)),
        ],
        out_shape=[
            jax.ShapeDtypeStruct((NP, D), jnp.float32),
            jax.ShapeDtypeStruct((1, D), jnp.float32),
        ],
    )(aggp, degd, b2, a2)


def kernel(feat, edge_index, W, b, prelu_a):
    src = edge_index[0].astype(jnp.int32)
    dst = edge_index[1].astype(jnp.int32)
    fill = jnp.full((EP - E,), N, jnp.int32)
    src_pad = jnp.concatenate([src, fill])
    dst_pad = jnp.concatenate([dst, fill])
    src3 = src_pad.reshape(NW, NCHUNK, CH)
    dst3 = dst_pad.reshape(NW, NCHUNK, CH)
    featp = jnp.pad(feat, ((0, NP - N), (0, 0)))

    zN = jnp.zeros((NP,), jnp.float32)

    degs, degd = _hist(src3, dst3, zN)
    x = _xw(featp, degs, W)
    aggp = _agg(x, src3, dst3)
    h_pad, hg = _final(aggp, degd, b.reshape(1, D),
                       jnp.reshape(prelu_a, (1, 1)))
    return h_pad[:N], hg


# speedup vs baseline: 1.2597x; 1.0229x over previous
import functools

import jax
import jax.numpy as jnp
from jax import lax
from jax.experimental import pallas as pl
from jax.experimental.pallas import tpu as pltpu
from jax.experimental.pallas import tpu_sc as plsc

N = 10000
E = 320000
D = 128
NP = 10240
NC = 2
NS = 16
NW = NC * NS
CH = 128
NCHUNK = 80
EPT = NCHUNK * CH
EP = NW * EPT
ROWS = NP // NS
BLK = 512
NBLK = NP // BLK

_mesh = plsc.VectorSubcoreMesh(core_axis_name="c", subcore_axis_name="s")


@functools.partial(
    pl.kernel,
    out_type=jax.ShapeDtypeStruct((NW, 2 * NP), jnp.float32),
    mesh=_mesh,
    scratch_types=[
        pltpu.VMEM((2, NCHUNK, CH), jnp.int32),
        pltpu.VMEM((2 * NP,), jnp.float32),
    ],
    compiler_params=pltpu.CompilerParams(needs_layout_passes=False),
)
def _hist(srcdst, z2N, deg_out, idx_v, cnt):
    c = lax.axis_index("c")
    s = lax.axis_index("s")
    g = s * NC + c
    pltpu.sync_copy(z2N, cnt)
    pltpu.sync_copy(srcdst.at[g], idx_v)
    ones = jnp.ones((16,), jnp.float32)

    def body(j, carry):
        def inner(k, carry2):
            iv_s = idx_v[0, j, pl.ds(k * 16, 16)]
            plsc.addupdate_scatter(cnt, [iv_s], ones)
            iv_d = idx_v[1, j, pl.ds(k * 16, 16)] + NP
            plsc.addupdate_scatter(cnt, [iv_d], ones)
            return carry2
        return lax.fori_loop(0, CH // 16, inner, carry)

    lax.fori_loop(0, NCHUNK, body, 0)
    pltpu.sync_copy(cnt, deg_out.at[g])


def _xw_body(feat_ref, degs_ref, w_ref, x_ref):
    d = jnp.sum(degs_ref[...], axis=0)[:, None]
    ns = jnp.where(d > 0, lax.rsqrt(d), 0.0)
    x_ref[...] = jnp.dot(feat_ref[...] * ns, w_ref[...],
                         preferred_element_type=jnp.float32)


def _xw(featp, degs, W):
    return pl.pallas_call(
        _xw_body,
        grid=(NBLK,),
        in_specs=[
            pl.BlockSpec((BLK, D), lambda i: (i, 0)),
            pl.BlockSpec((NW, BLK), lambda i: (0, i)),
            pl.BlockSpec((D, D), lambda i: (0, 0)),
        ],
        out_specs=pl.BlockSpec((BLK, D), lambda i: (i, 0)),
        out_shape=jax.ShapeDtypeStruct((NP, D), jnp.float32),
    )(featp, degs, W)


ST = 40


@functools.partial(
    pl.kernel,
    out_type=jax.ShapeDtypeStruct((NC, NP, D), jnp.float32),
    mesh=_mesh,
    scratch_types=[
        pltpu.VMEM((ST, CH), jnp.int32),
        pltpu.VMEM((ST, CH), jnp.int32),
        pltpu.VMEM((CH, D), jnp.float32),
        pltpu.VMEM((CH, D), jnp.float32),
        pltpu.VMEM_SHARED((NP, D), jnp.float32),
        pltpu.SemaphoreType.DMA,
        pltpu.SemaphoreType.DMA,
    ],
)
def _agg(x, src3, dst3, zD, out,
         sidx, didx, buf0, buf1, agg_sh, sem0, sem1):
    c = lax.axis_index("c")
    s = lax.axis_index("s")
    g = s * NC + c
    pltpu.sync_copy(zD.at[pl.ds(s * ROWS, ROWS)],
                    agg_sh.at[pl.ds(s * ROWS, ROWS)])
    plsc.subcore_barrier()

    for p in range(NCHUNK // ST):
        pltpu.sync_copy(src3.at[g, pl.ds(p * ST, ST)], sidx)
        pltpu.sync_copy(dst3.at[g, pl.ds(p * ST, ST)], didx)

        pltpu.async_copy(x.at[sidx.at[0]], buf0, sem0)

        def body(t, carry):
            j0 = 2 * t
            pltpu.async_copy(x.at[sidx.at[j0 + 1]], buf1, sem1)
            pltpu.make_async_copy(x.at[sidx.at[j0]], buf0, sem0).wait()
            pltpu.sync_copy(buf0, agg_sh.at[didx.at[j0]], add=True)
            pltpu.async_copy(x.at[sidx.at[j0 + 2]], buf0, sem0)
            pltpu.make_async_copy(x.at[sidx.at[j0 + 1]], buf1, sem1).wait()
            pltpu.sync_copy(buf1, agg_sh.at[didx.at[j0 + 1]], add=True)
            return carry

        lax.fori_loop(0, ST // 2 - 1, body, 0)

        jl = ST - 2
        pltpu.async_copy(x.at[sidx.at[jl + 1]], buf1, sem1)
        pltpu.make_async_copy(x.at[sidx.at[jl]], buf0, sem0).wait()
        pltpu.sync_copy(buf0, agg_sh.at[didx.at[jl]], add=True)
        pltpu.make_async_copy(x.at[sidx.at[jl + 1]], buf1, sem1).wait()
        pltpu.sync_copy(buf1, agg_sh.at[didx.at[jl + 1]], add=True)

    plsc.subcore_barrier()
    for k in range(ROWS // CH):
        so = s * ROWS + k * CH
        pltpu.sync_copy(agg_sh.at[pl.ds(so, CH)], out.at[c, pl.ds(so, CH)])


def _final_body(aggp_ref, degd_ref, b_ref, a_ref, h_ref, hg_ref):
    i = pl.program_id(0)
    agg = aggp_ref[0] + aggp_ref[1]
    d = jnp.sum(degd_ref[...], axis=0)[:, None]
    nd = jnp.where(d > 0, lax.rsqrt(d), 0.0)
    rst = agg * nd + b_ref[...]
    a = a_ref[0, 0]
    h = jnp.maximum(rst, 0.0) + a * jnp.minimum(rst, 0.0)
    row = i * BLK + lax.broadcasted_iota(jnp.int32, (BLK, 1), 0)
    h = jnp.where(row < N, h, 0.0)
    h_ref[...] = h

    @pl.when(i == 0)
    def _init():
        hg_ref[...] = jnp.zeros_like(hg_ref)

    hg_ref[...] += jnp.sum(h, axis=0, keepdims=True)


def _final(aggp, degd, b2, a2):
    return pl.pallas_call(
        _final_body,
        grid=(NBLK,),
        in_specs=[
            pl.BlockSpec((NC, BLK, D), lambda i: (0, i, 0)),
            pl.BlockSpec((NW, BLK), lambda i: (0, i)),
            pl.BlockSpec((1, D), lambda i: (0, 0)),
            pl.BlockSpec((1, 1), lambda i: (0, 0)),
        ],
        out_specs=[
            pl.BlockSpec((BLK, D), lambda i: (i, 0)),
            pl.BlockSpec((1, D), lambda i: (0, 0)),
        ],
        out_shape=[
            jax.ShapeDtypeStruct((NP, D), jnp.float32),
            jax.ShapeDtypeStruct((1, D), jnp.float32),
        ],
    )(aggp, degd, b2, a2)


def kernel(feat, edge_index, W, b, prelu_a):
    src = edge_index[0].astype(jnp.int32)
    dst = edge_index[1].astype(jnp.int32)
    fill = jnp.full((EP - E,), N, jnp.int32)
    src_pad = jnp.concatenate([src, fill])
    dst_pad = jnp.concatenate([dst, fill])
    src3 = src_pad.reshape(NW, NCHUNK, CH)
    dst3 = dst_pad.reshape(NW, NCHUNK, CH)
    featp = jnp.pad(feat, ((0, NP - N), (0, 0)))

    z2N = jnp.zeros((2 * NP,), jnp.float32)
    zD = jnp.zeros((NP, D), jnp.float32)
    srcdst = jnp.stack([src3, dst3], axis=1)

    deg_all = _hist(srcdst, z2N).reshape(NW, 2, NP)
    degs = deg_all[:, 0]
    degd = deg_all[:, 1]
    x = _xw(featp, degs, W)
    aggp = _agg(x, src3, dst3, zD)
    h_pad, hg = _final(aggp, degd, b.reshape(1, D),
                       jnp.reshape(prelu_a, (1, 1)))
    return h_pad[:N], hg


# speedup vs baseline: 1.2605x; 1.0006x over previous
import functools

import jax
import jax.numpy as jnp
from jax import lax
from jax.experimental import pallas as pl
from jax.experimental.pallas import tpu as pltpu
from jax.experimental.pallas import tpu_sc as plsc

N = 10000
E = 320000
D = 128
NP = 10240
NC = 2
NS = 16
NW = NC * NS
CH = 128
NCHUNK = 80
EPT = NCHUNK * CH
EP = NW * EPT
ROWS = NP // NS
BLK = 512
NBLK = NP // BLK

_mesh = plsc.VectorSubcoreMesh(core_axis_name="c", subcore_axis_name="s")


@functools.partial(
    pl.kernel,
    out_type=jax.ShapeDtypeStruct((NW, 2 * NP), jnp.float32),
    mesh=_mesh,
    scratch_types=[
        pltpu.VMEM((2, NCHUNK, CH), jnp.int32),
        pltpu.VMEM((2 * NP,), jnp.float32),
    ],
    compiler_params=pltpu.CompilerParams(needs_layout_passes=False),
)
def _hist(srcdst, z2N, deg_out, idx_v, cnt):
    c = lax.axis_index("c")
    s = lax.axis_index("s")
    g = s * NC + c
    pltpu.sync_copy(z2N, cnt)
    pltpu.sync_copy(srcdst.at[g], idx_v)
    ones = jnp.ones((16,), jnp.float32)

    def body(j, carry):
        def inner(k, carry2):
            iv_s = idx_v[0, j, pl.ds(k * 16, 16)]
            plsc.addupdate_scatter(cnt, [iv_s], ones)
            iv_d = idx_v[1, j, pl.ds(k * 16, 16)] + NP
            plsc.addupdate_scatter(cnt, [iv_d], ones)
            return carry2
        return lax.fori_loop(0, CH // 16, inner, carry)

    lax.fori_loop(0, NCHUNK, body, 0)
    pltpu.sync_copy(cnt, deg_out.at[g])


def _xw_body(feat_ref, degs_ref, w_ref, x_ref):
    d = jnp.sum(degs_ref[...], axis=0)[:, None]
    ns = jnp.where(d > 0, lax.rsqrt(d), 0.0)
    x_ref[...] = jnp.dot(feat_ref[...] * ns, w_ref[...],
                         preferred_element_type=jnp.float32)


def _xw(featp, degs, W):
    return pl.pallas_call(
        _xw_body,
        grid=(NBLK,),
        in_specs=[
            pl.BlockSpec((BLK, D), lambda i: (i, 0)),
            pl.BlockSpec((NW, BLK), lambda i: (0, i)),
            pl.BlockSpec((D, D), lambda i: (0, 0)),
        ],
        out_specs=pl.BlockSpec((BLK, D), lambda i: (i, 0)),
        out_shape=jax.ShapeDtypeStruct((NP, D), jnp.float32),
    )(featp, degs, W)


ST = 40


@functools.partial(
    pl.kernel,
    out_type=jax.ShapeDtypeStruct((NC, NP, D), jnp.float32),
    mesh=_mesh,
    scratch_types=[
        pltpu.VMEM((ST, CH), jnp.int32),
        pltpu.VMEM((ST, CH), jnp.int32),
        pltpu.VMEM((CH, D), jnp.float32),
        pltpu.VMEM((CH, D), jnp.float32),
        pltpu.VMEM_SHARED((NP, D), jnp.float32),
        pltpu.SemaphoreType.DMA,
        pltpu.SemaphoreType.DMA,
    ],
)
def _agg(x, src3, dst3, zD, out,
         sidx, didx, buf0, buf1, agg_sh, sem0, sem1):
    c = lax.axis_index("c")
    s = lax.axis_index("s")
    g = s * NC + c
    pltpu.sync_copy(zD.at[pl.ds(s * ROWS, ROWS)],
                    agg_sh.at[pl.ds(s * ROWS, ROWS)])
    plsc.subcore_barrier()

    for p in range(NCHUNK // ST):
        pltpu.sync_copy(src3.at[g, pl.ds(p * ST, ST)], sidx)
        pltpu.sync_copy(dst3.at[g, pl.ds(p * ST, ST)], didx)

        pltpu.async_copy(x.at[sidx.at[0]], buf0, sem0)

        def body(t, carry):
            j0 = 2 * t
            pltpu.async_copy(x.at[sidx.at[j0 + 1]], buf1, sem1)
            pltpu.make_async_copy(x.at[sidx.at[j0]], buf0, sem0).wait()
            pltpu.sync_copy(buf0, agg_sh.at[didx.at[j0]], add=True)
            pltpu.async_copy(x.at[sidx.at[j0 + 2]], buf0, sem0)
            pltpu.make_async_copy(x.at[sidx.at[j0 + 1]], buf1, sem1).wait()
            pltpu.sync_copy(buf1, agg_sh.at[didx.at[j0 + 1]], add=True)
            return carry

        lax.fori_loop(0, ST // 2 - 1, body, 0)

        jl = ST - 2
        pltpu.async_copy(x.at[sidx.at[jl + 1]], buf1, sem1)
        pltpu.make_async_copy(x.at[sidx.at[jl]], buf0, sem0).wait()
        pltpu.sync_copy(buf0, agg_sh.at[didx.at[jl]], add=True)
        pltpu.make_async_copy(x.at[sidx.at[jl + 1]], buf1, sem1).wait()
        pltpu.sync_copy(buf1, agg_sh.at[didx.at[jl + 1]], add=True)

    plsc.subcore_barrier()
    pltpu.sync_copy(agg_sh.at[pl.ds(s * ROWS, ROWS)],
                    out.at[c, pl.ds(s * ROWS, ROWS)])


def _final_body(aggp_ref, degd_ref, b_ref, a_ref, h_ref, hg_ref):
    i = pl.program_id(0)
    agg = aggp_ref[0] + aggp_ref[1]
    d = jnp.sum(degd_ref[...], axis=0)[:, None]
    nd = jnp.where(d > 0, lax.rsqrt(d), 0.0)
    rst = agg * nd + b_ref[...]
    a = a_ref[0, 0]
    h = jnp.maximum(rst, 0.0) + a * jnp.minimum(rst, 0.0)
    row = i * BLK + lax.broadcasted_iota(jnp.int32, (BLK, 1), 0)
    h = jnp.where(row < N, h, 0.0)
    h_ref[...] = h

    @pl.when(i == 0)
    def _init():
        hg_ref[...] = jnp.zeros_like(hg_ref)

    hg_ref[...] += jnp.sum(h, axis=0, keepdims=True)


def _final(aggp, degd, b2, a2):
    return pl.pallas_call(
        _final_body,
        grid=(NBLK,),
        in_specs=[
            pl.BlockSpec((NC, BLK, D), lambda i: (0, i, 0)),
            pl.BlockSpec((NW, BLK), lambda i: (0, i)),
            pl.BlockSpec((1, D), lambda i: (0, 0)),
            pl.BlockSpec((1, 1), lambda i: (0, 0)),
        ],
        out_specs=[
            pl.BlockSpec((BLK, D), lambda i: (i, 0)),
            pl.BlockSpec((1, D), lambda i: (0, 0)),
        ],
        out_shape=[
            jax.ShapeDtypeStruct((NP, D), jnp.float32),
            jax.ShapeDtypeStruct((1, D), jnp.float32),
        ],
    )(aggp, degd, b2, a2)


def kernel(feat, edge_index, W, b, prelu_a):
    src = edge_index[0].astype(jnp.int32)
    dst = edge_index[1].astype(jnp.int32)
    fill = jnp.full((EP - E,), N, jnp.int32)
    src_pad = jnp.concatenate([src, fill])
    dst_pad = jnp.concatenate([dst, fill])
    src3 = src_pad.reshape(NW, NCHUNK, CH)
    dst3 = dst_pad.reshape(NW, NCHUNK, CH)
    featp = jnp.pad(feat, ((0, NP - N), (0, 0)))

    z2N = jnp.zeros((2 * NP,), jnp.float32)
    zD = jnp.zeros((NP, D), jnp.float32)
    srcdst = jnp.stack([src3, dst3], axis=1)

    deg_all = _hist(srcdst, z2N).reshape(NW, 2, NP)
    degs = deg_all[:, 0]
    degd = deg_all[:, 1]
    x = _xw(featp, degs, W)
    aggp = _agg(x, src3, dst3, zD)
    h_pad, hg = _final(aggp, degd, b.reshape(1, D),
                       jnp.reshape(prelu_a, (1, 1)))
    return h_pad[:N], hg
